# Initial kernel scaffold; baseline (speedup 1.0000x reference)
import jax
import jax.numpy as jnp
from jax.experimental import pallas as pl


def kernel(x, table):
    raise NotImplementedError("write your pallas kernel here")


# speedup vs baseline: 1.0946x; 1.0946x over previous
import functools

import jax
import jax.numpy as jnp
from jax import lax
from jax.experimental import pallas as pl
from jax.experimental.pallas import tpu as pltpu
from jax.experimental.pallas import tpu_sc as plsc

_CHUNK = 128
_NBUF = 2
_LANES = 16


@functools.lru_cache(maxsize=None)
def _build_gather(B, V, D):
    info = plsc.get_sparse_core_info()
    nw = info.num_cores * info.num_subcores
    assert B % nw == 0
    b_per_w = B // nw
    assert b_per_w % (_CHUNK * _NBUF) == 0
    nchunks = b_per_w // _CHUNK
    niter = nchunks // _NBUF

    mesh = plsc.VectorSubcoreMesh(core_axis_name="c", subcore_axis_name="s")

    @functools.partial(
        pl.kernel,
        mesh=mesh,
        out_type=jax.ShapeDtypeStruct((B, D), jnp.float32),
        scratch_types=[
            pltpu.VMEM((b_per_w,), jnp.int32),
            pltpu.VMEM((_CHUNK, D), jnp.float32),
            pltpu.VMEM((_CHUNK, D), jnp.float32),
            pltpu.SemaphoreType.DMA,
            pltpu.SemaphoreType.DMA,
            pltpu.SemaphoreType.DMA,
            pltpu.SemaphoreType.DMA,
        ],
    )
    def body(x_hbm, table_hbm, out_hbm, idx_v, rows0, rows1, g0, g1, s0, s1):
        wid = lax.axis_index("s") * info.num_cores + lax.axis_index("c")
        base = wid * b_per_w
        pltpu.sync_copy(x_hbm.at[pl.ds(base, b_per_w)], idx_v)

        rows = (rows0, rows1)
        gsem = (g0, g1)
        ssem = (s0, s1)

        def fire_gather(c, b):
            def grp_body(g, carry):
                vv = idx_v[pl.ds(c * _CHUNK + g * _LANES, _LANES)]
                for k in range(_LANES):
                    r = jnp.squeeze(lax.slice(vv, (k,), (k + 1,)))
                    pltpu.async_copy(table_hbm.at[pl.ds(r, 1)],
                                     rows[b].at[pl.ds(g * _LANES + k, 1)],
                                     gsem[b])
                return carry

            lax.fori_loop(0, _CHUNK // _LANES, grp_body, 0)

        def wait_gather(b):
            def row_body(k, carry):
                pltpu.make_async_copy(table_hbm.at[pl.ds(0, 1)],
                                      rows[b].at[pl.ds(0, 1)],
                                      gsem[b]).wait()
                return carry

            lax.fori_loop(0, _CHUNK, row_body, 0)

        def fire_scatter(c, b):
            pltpu.async_copy(rows[b],
                             out_hbm.at[pl.ds(base + c * _CHUNK, _CHUNK)],
                             ssem[b])

        def wait_scatter(b):
            pltpu.make_async_copy(rows[b],
                                  out_hbm.at[pl.ds(base, _CHUNK)],
                                  ssem[b]).wait()

        for b in range(_NBUF):
            fire_gather(b, b)

        def iter_body(i, carry):
            for b in range(_NBUF):
                c = _NBUF * i + b
                wait_gather(b)
                fire_scatter(c, b)
                wait_scatter(b)
                fire_gather(c + _NBUF, b)
            return carry

        lax.fori_loop(0, niter - 1, iter_body, 0)

        for b in range(_NBUF):
            wait_gather(b)
            fire_scatter(_NBUF * (niter - 1) + b, b)
        for b in range(_NBUF):
            wait_scatter(b)

    return body


def kernel(x, table):
    V, D = table.shape
    out = _build_gather(x.size, V, D)(x.reshape(x.size), table)
    return out.reshape(x.shape + (D,))


# speedup vs baseline: 1.2863x; 1.1752x over previous
import functools

import jax
import jax.numpy as jnp
from jax import lax
from jax.experimental import pallas as pl
from jax.experimental.pallas import tpu as pltpu
from jax.experimental.pallas import tpu_sc as plsc

_SPC = 2
_NBUF = 2
_LANES = 16


@functools.lru_cache(maxsize=None)
def _build_gather(NB, SEQ, V, D):
    info = plsc.get_sparse_core_info()
    nw = info.num_cores * info.num_subcores
    assert NB % (nw * _SPC * _NBUF) == 0
    s_per_w = NB // nw
    b_per_w = s_per_w * SEQ
    rows_c = _SPC * SEQ
    nchunks = s_per_w // _SPC
    niter = nchunks // _NBUF
    ngrp = rows_c // _LANES
    tail = rows_c - ngrp * _LANES

    mesh = plsc.VectorSubcoreMesh(core_axis_name="c", subcore_axis_name="s")

    @functools.partial(
        pl.kernel,
        mesh=mesh,
        out_type=jax.ShapeDtypeStruct((NB, SEQ, D), jnp.float32),
        scratch_types=[
            pltpu.VMEM((b_per_w + _LANES,), jnp.int32),
            pltpu.VMEM((_SPC, SEQ, D), jnp.float32),
            pltpu.VMEM((_SPC, SEQ, D), jnp.float32),
            pltpu.SemaphoreType.DMA,
            pltpu.SemaphoreType.DMA,
            pltpu.SemaphoreType.DMA,
            pltpu.SemaphoreType.DMA,
        ],
    )
    def body(x_hbm, table_hbm, out_hbm, idx_v, rows0, rows1, g0, g1, s0, s1):
        wid = lax.axis_index("s") * info.num_cores + lax.axis_index("c")
        base = wid * b_per_w
        pltpu.sync_copy(x_hbm.at[pl.ds(base, b_per_w)],
                        idx_v.at[pl.ds(0, b_per_w)])

        rows = (rows0, rows1)
        gsem = (g0, g1)
        ssem = (s0, s1)

        def enqueue(vv, j, dst_row, b):
            r = jnp.squeeze(lax.slice(vv, (j,), (j + 1,)))
            q = dst_row // SEQ
            s = dst_row - q * SEQ
            pltpu.async_copy(table_hbm.at[pl.ds(r, 1)],
                             rows[b].at[q].at[pl.ds(s, 1)], gsem[b])

        def fire_gather(c, b):
            def grp_body(g, carry):
                vv = idx_v[pl.ds(c * rows_c + g * _LANES, _LANES)]
                for j in range(_LANES):
                    enqueue(vv, j, g * _LANES + j, b)
                return carry

            lax.fori_loop(0, ngrp, grp_body, 0)
            if tail:
                vv = idx_v[pl.ds(c * rows_c + ngrp * _LANES, _LANES)]
                for j in range(tail):
                    enqueue(vv, j, ngrp * _LANES + j, b)

        def wait_gather(b):
            def row_body(k, carry):
                pltpu.make_async_copy(table_hbm.at[pl.ds(0, 1)],
                                      rows[b].at[0].at[pl.ds(0, 1)],
                                      gsem[b]).wait()
                return carry

            lax.fori_loop(0, rows_c, row_body, 0)

        def fire_scatter(c, b):
            s0_ = wid * s_per_w + c * _SPC
            for q in range(_SPC):
                pltpu.async_copy(rows[b].at[q], out_hbm.at[s0_ + q], ssem[b])

        def wait_scatter(b):
            for q in range(_SPC):
                pltpu.make_async_copy(rows[b].at[0], out_hbm.at[0],
                                      ssem[b]).wait()

        for b in range(_NBUF):
            fire_gather(b, b)

        def iter_body(i, carry):
            for b in range(_NBUF):
                c = _NBUF * i + b
                wait_gather(b)
                fire_scatter(c, b)
                wait_scatter(b)
                fire_gather(c + _NBUF, b)
            return carry

        lax.fori_loop(0, niter - 1, iter_body, 0)

        for b in range(_NBUF):
            wait_gather(b)
            fire_scatter(_NBUF * (niter - 1) + b, b)
        for b in range(_NBUF):
            wait_scatter(b)

    return body


def kernel(x, table):
    NB, SEQ = x.shape
    V, D = table.shape
    return _build_gather(NB, SEQ, V, D)(x.reshape(x.size), table)
